# baseline (speedup 1.0000x reference)
import functools

import jax
import jax.numpy as jnp
from jax import lax
from jax.experimental import pallas as pl
from jax.experimental.pallas import tpu as pltpu
from jax.experimental.pallas import tpu_sc as plsc

BATCH = 16384
EMBED_DIM = 16
N_ROWS = 1000001

_info = plsc.get_sparse_core_info()
_NC, _NS = _info.num_cores, _info.num_subcores
_NW = _NC * _NS

_CLS_PER_W = 32768
_WIN = 2048
_NWIN = _CLS_PER_W // _WIN
_C0_MAX = 998016
_CAP = 1024
_OUT_ROWS = BATCH + 8
_DUMP_ROW = BATCH


def _emb_kernel(wt_hbm, idx_hbm, out_hbm, idx_v, b_l, r_l, wb2d, wr_l, buf0,
                buf1, st1, st2a, st2b, sem0, sem1, sem2a, sem2b):
    wid = lax.axis_index("s") * _NC + lax.axis_index("c")
    lo = wid * _CLS_PER_W
    iota16 = lax.iota(jnp.int32, 16)

    bufs = (buf0, buf1)
    sems = (sem0, sem1)

    def fire(i):
        c0 = jnp.minimum(lo + i * _WIN, _C0_MAX)
        return pltpu.async_copy(
            wt_hbm.at[:, :, pl.ds(c0, _WIN)], bufs[i % 2], sems[i % 2])

    cp = fire(0)
    pltpu.sync_copy(idx_hbm, idx_v)

    def scan_body(j, cur_v):
        rv = idx_v[pl.ds(j * 16, 16)]
        m = (rv >= lo) & (rv < lo + _CLS_PER_W)
        bv = j * 16 + iota16
        mi = jnp.where(m, jnp.int32(1), jnp.int32(0))
        pos = jnp.minimum(cur_v + jnp.cumsum(mi) - 1, _CAP - 1)
        plsc.store_scatter(r_l, [pos], rv, mask=m)
        plsc.store_scatter(b_l, [pos], bv, mask=m)
        return cur_v + plsc.all_reduce_population_count(m)

    n_t_v = lax.fori_loop(0, BATCH // 16, scan_body,
                          jnp.zeros((16,), jnp.int32), unroll=4)
    n_t = lax.squeeze(lax.slice(n_t_v, [0], [1]), [0])

    for i in range(_NWIN):
        nxt = fire(i + 1) if i + 1 < _NWIN else None
        cp.wait()
        buf = bufs[i % 2]
        c0 = jnp.minimum(lo + i * _WIN, _C0_MAX)

        def filt_body(g, wcur_v, i=i):
            rv = r_l[pl.ds(g * 16, 16)]
            bv = b_l[pl.ds(g * 16, 16)]
            valid = (g * 16 + iota16) < n_t
            wm = valid & (((rv - lo) >> 11) == i)
            wmi = jnp.where(wm, jnp.int32(1), jnp.int32(0))
            pos = jnp.minimum(wcur_v + jnp.cumsum(wmi) - 1, _CAP - 1)
            plsc.store_scatter(wr_l, [pos], rv, mask=wm)
            plsc.store_scatter(wb2d, [pos >> 4, pos & 15], bv, mask=wm)
            return wcur_v + plsc.all_reduce_population_count(wm)

        n_w_v = lax.fori_loop(0, (n_t + 15) >> 4, filt_body,
                              jnp.zeros((16,), jnp.int32))
        n_w = lax.squeeze(lax.slice(n_w_v, [0], [1]), [0])

        st2s = (st2a, st2b)
        sems2 = (sem2a, sem2b)

        def gather_pair(h, _, buf=buf, c0=c0):
            for k in range(2):
                g = 2 * h + k
                st2 = st2s[k]
                sem2 = sems2[k]

                def _drain(st2=st2, sem2=sem2):
                    pltpu.make_async_copy(
                        out_hbm.at[pl.ds(0, 16)], st2, sem2).wait()

                pl.when(h >= 1)(_drain)
                rv = wr_l[pl.ds(g * 16, 16)]
                tail = (g * 16 + iota16) < n_w
                col = jnp.where(tail, rv - c0, 0)
                for d in range(EMBED_DIM):
                    st1[d, :] = plsc.load_gather(
                        buf,
                        [jnp.full((16,), d >> 3, jnp.int32),
                         jnp.full((16,), d & 7, jnp.int32), col])
                for j in range(16):
                    st2[j, 0:16] = plsc.load_gather(
                        st1, [iota16, jnp.full((16,), j, jnp.int32)])
                brow = wb2d[g, :]
                wb2d[g, :] = jnp.where(tail, brow, _DUMP_ROW)
                pltpu.async_copy(st2, out_hbm.at[wb2d.at[g]], sem2)
            return 0

        n_pairs = (n_w + 31) >> 5
        lax.fori_loop(0, n_pairs, gather_pair, 0)

        def _final_drain():
            for k in range(2):
                pltpu.make_async_copy(
                    out_hbm.at[pl.ds(0, 16)], st2s[k], sems2[k]).wait()

        pl.when(n_pairs >= 1)(_final_drain)
        cp = nxt


@jax.jit
def _embed_lookup(table_t3, idx):
    mesh = plsc.VectorSubcoreMesh(core_axis_name="c", subcore_axis_name="s")
    return pl.kernel(
        _emb_kernel,
        mesh=mesh,
        out_type=jax.ShapeDtypeStruct((_OUT_ROWS, 128), jnp.float32),
        scratch_types=[
            pltpu.VMEM((BATCH,), jnp.int32),
            pltpu.VMEM((_CAP,), jnp.int32),
            pltpu.VMEM((_CAP,), jnp.int32),
            pltpu.VMEM((_CAP // 16, 16), jnp.int32),
            pltpu.VMEM((_CAP,), jnp.int32),
            pltpu.VMEM((2, 8, _WIN), jnp.float32),
            pltpu.VMEM((2, 8, _WIN), jnp.float32),
            pltpu.VMEM((16, 16), jnp.float32),
            pltpu.VMEM((16, 128), jnp.float32),
            pltpu.VMEM((16, 128), jnp.float32),
            pltpu.SemaphoreType.DMA,
            pltpu.SemaphoreType.DMA,
            pltpu.SemaphoreType.DMA,
            pltpu.SemaphoreType.DMA,
        ],
        compiler_params=pltpu.CompilerParams(
            disable_bounds_checks=True, needs_layout_passes=False),
    )(table_t3, idx)


def kernel(class_label, embedding_weight):
    wt3 = embedding_weight.T.reshape(2, 8, N_ROWS)
    out = _embed_lookup(wt3, class_label)
    return out[:BATCH, :EMBED_DIM][:, None, :]


# speedup vs baseline: 1.0408x; 1.0408x over previous
import functools

import jax
import jax.numpy as jnp
from jax import lax
from jax.experimental import pallas as pl
from jax.experimental.pallas import tpu as pltpu
from jax.experimental.pallas import tpu_sc as plsc

BATCH = 16384
EMBED_DIM = 16
N_ROWS = 1000001

_info = plsc.get_sparse_core_info()
_NC, _NS = _info.num_cores, _info.num_subcores
_NW = _NC * _NS

_CLS_PER_W = 32768
_WIN = 2048
_NWIN = _CLS_PER_W // _WIN
_C0_MAX = 998016
_CAP = 1024
_CAP_W = 64
_INTER_ROWS = _NW * _NWIN * _CAP_W
_B_PER_W = BATCH // _NW


def _stream_kernel(wt_hbm, idx_hbm, inter_hbm, idx_v, b_l, r_l, wb_l, wr_l,
                   buf0, buf1, st1, st2, sem0, sem1, sem2):
    wid = lax.axis_index("s") * _NC + lax.axis_index("c")
    lo = wid * _CLS_PER_W
    iota16 = lax.iota(jnp.int32, 16)

    bufs = (buf0, buf1)
    sems = (sem0, sem1)

    def fire(i):
        c0 = jnp.minimum(lo + i * _WIN, _C0_MAX)
        return pltpu.async_copy(
            wt_hbm.at[:, :, pl.ds(c0, _WIN)], bufs[i % 2], sems[i % 2])

    cp = fire(0)
    pltpu.sync_copy(idx_hbm, idx_v)

    def scan_body(j, cur_v):
        rv = idx_v[pl.ds(j * 16, 16)]
        m = (rv >= lo) & (rv < lo + _CLS_PER_W)
        bv = j * 16 + iota16
        mi = jnp.where(m, jnp.int32(1), jnp.int32(0))
        pos = jnp.minimum(cur_v + jnp.cumsum(mi) - 1, _CAP - 1)
        plsc.store_scatter(r_l, [pos], rv, mask=m)
        plsc.store_scatter(b_l, [pos], bv, mask=m)
        return cur_v + plsc.all_reduce_population_count(m)

    n_t_v = lax.fori_loop(0, BATCH // 16, scan_body,
                          jnp.zeros((16,), jnp.int32), unroll=4)
    n_t = lax.squeeze(lax.slice(n_t_v, [0], [1]), [0])

    for i in range(_NWIN):
        nxt = fire(i + 1) if i + 1 < _NWIN else None
        cp.wait()
        buf = bufs[i % 2]
        c0 = jnp.minimum(lo + i * _WIN, _C0_MAX)

        def filt_body(g, wcur_v, i=i):
            rv = r_l[pl.ds(g * 16, 16)]
            valid = (g * 16 + iota16) < n_t
            wm = valid & (((rv - lo) >> 11) == i)
            wmi = jnp.where(wm, jnp.int32(1), jnp.int32(0))
            pos = jnp.minimum(wcur_v + jnp.cumsum(wmi) - 1, _CAP_W - 1)
            plsc.store_scatter(wr_l, [pos], rv, mask=wm)
            return wcur_v + plsc.all_reduce_population_count(wm)

        n_w_v = lax.fori_loop(0, (n_t + 15) >> 4, filt_body,
                              jnp.zeros((16,), jnp.int32))
        n_w = jnp.minimum(
            lax.squeeze(lax.slice(n_w_v, [0], [1]), [0]), _CAP_W)
        row0 = wid * (_NWIN * _CAP_W) + i * _CAP_W

        def gather_body(g2, _, buf=buf, c0=c0, row0=row0):
            rv = wr_l[pl.ds(g2 * 16, 16)]
            tail = (g2 * 16 + iota16) < n_w
            col = jnp.where(tail, rv - c0, 0)
            for d in range(EMBED_DIM):
                st1[d, :] = plsc.load_gather(
                    buf,
                    [jnp.full((16,), d >> 3, jnp.int32),
                     jnp.full((16,), d & 7, jnp.int32), col])
            for j in range(16):
                st2[j, 0:16] = plsc.load_gather(
                    st1, [iota16, jnp.full((16,), j, jnp.int32)])
            pltpu.async_copy(
                st2, inter_hbm.at[pl.ds(row0 + g2 * 16, 16)], sem2).wait()
            return 0

        lax.fori_loop(0, (n_w + 15) >> 4, gather_body, 0)
        cp = nxt


def _reorder_kernel(inter_hbm, posn_hbm, out_hbm, pos_v, rows_v, out_v, sem):
    wid = lax.axis_index("s") * _NC + lax.axis_index("c")
    b0 = wid * _B_PER_W
    pltpu.sync_copy(posn_hbm.at[pl.ds(b0, _B_PER_W)], pos_v)
    pltpu.async_copy(inter_hbm.at[pos_v], rows_v, sem).wait()

    def pick_body(j, _):
        out_v[j, :] = rows_v[j, 0:16]
        return 0

    lax.fori_loop(0, _B_PER_W, pick_body, 0, unroll=8)
    pltpu.sync_copy(out_v, out_hbm.at[pl.ds(b0, _B_PER_W)])


@jax.jit
def _embed_lookup(table_t3, idx, positions):
    mesh = plsc.VectorSubcoreMesh(core_axis_name="c", subcore_axis_name="s")
    inter = pl.kernel(
        _stream_kernel,
        mesh=mesh,
        out_type=jax.ShapeDtypeStruct((_INTER_ROWS, 128), jnp.float32),
        scratch_types=[
            pltpu.VMEM((BATCH,), jnp.int32),
            pltpu.VMEM((_CAP,), jnp.int32),
            pltpu.VMEM((_CAP,), jnp.int32),
            pltpu.VMEM((_CAP,), jnp.int32),
            pltpu.VMEM((_CAP,), jnp.int32),
            pltpu.VMEM((2, 8, _WIN), jnp.float32),
            pltpu.VMEM((2, 8, _WIN), jnp.float32),
            pltpu.VMEM((16, 16), jnp.float32),
            pltpu.VMEM((16, 128), jnp.float32),
            pltpu.SemaphoreType.DMA,
            pltpu.SemaphoreType.DMA,
            pltpu.SemaphoreType.DMA,
        ],
        compiler_params=pltpu.CompilerParams(
            disable_bounds_checks=True, needs_layout_passes=False),
    )(table_t3, idx)
    out = pl.kernel(
        _reorder_kernel,
        mesh=plsc.VectorSubcoreMesh(core_axis_name="c", subcore_axis_name="s"),
        out_type=jax.ShapeDtypeStruct((BATCH, EMBED_DIM), jnp.float32),
        scratch_types=[
            pltpu.VMEM((_B_PER_W,), jnp.int32),
            pltpu.VMEM((_B_PER_W, 128), jnp.float32),
            pltpu.VMEM((_B_PER_W, EMBED_DIM), jnp.float32),
            pltpu.SemaphoreType.DMA,
        ],
        compiler_params=pltpu.CompilerParams(
            disable_bounds_checks=True,
            needs_layout_passes=False,
            use_tc_tiling_on_sc=False,
        ),
    )(inter, positions)
    return out


def kernel(class_label, embedding_weight):
    wt3 = embedding_weight.T.reshape(2, 8, N_ROWS)
    grp = class_label >> 11
    oh = jax.nn.one_hot(grp, _NW * _NWIN, dtype=jnp.int32)
    ranks = jnp.take_along_axis(
        jnp.cumsum(oh, axis=0), grp[:, None], axis=1)[:, 0] - 1
    positions = grp * _CAP_W + jnp.minimum(ranks, _CAP_W - 1)
    out = _embed_lookup(wt3, class_label, positions.astype(jnp.int32))
    return out[:, None, :]


# speedup vs baseline: 1.6610x; 1.5959x over previous
import functools

import jax
import jax.numpy as jnp
from jax import lax
from jax.experimental import pallas as pl
from jax.experimental.pallas import tpu as pltpu
from jax.experimental.pallas import tpu_sc as plsc

BATCH = 16384
EMBED_DIM = 16
N_ROWS = 1000001

_info = plsc.get_sparse_core_info()
_NC, _NS = _info.num_cores, _info.num_subcores
_NW = _NC * _NS

_CLS_PER_W = 32768
_WIN = 2048
_NWIN = _CLS_PER_W // _WIN
_C0_MAX = 998016
_CAP = 1024
_OUT_ROWS = BATCH + 8
_DUMP_ROW = BATCH


def _emb_kernel(wt_hbm, idx_hbm, out_hbm, idx_v, b_l, r_l, wb_l, wr_l, buf0,
                buf1, st1, st2, sem0, sem1, sem2):
    wid = lax.axis_index("s") * _NC + lax.axis_index("c")
    lo = wid * _CLS_PER_W
    iota16 = lax.iota(jnp.int32, 16)

    bufs = (buf0, buf1)
    sems = (sem0, sem1)

    def fire(i):
        c0 = jnp.minimum(lo + i * _WIN, _C0_MAX)
        return pltpu.async_copy(
            wt_hbm.at[:, :, pl.ds(c0, _WIN)], bufs[i % 2], sems[i % 2])

    cp = fire(0)
    pltpu.sync_copy(idx_hbm, idx_v)

    def scan_body(j, cur_v):
        rv = idx_v[pl.ds(j * 16, 16)]
        m = (rv >= lo) & (rv < lo + _CLS_PER_W)
        bv = j * 16 + iota16
        mi = jnp.where(m, jnp.int32(1), jnp.int32(0))
        pos = jnp.minimum(cur_v + jnp.cumsum(mi) - 1, _CAP - 1)
        plsc.store_scatter(r_l, [pos], rv, mask=m)
        plsc.store_scatter(b_l, [pos], bv, mask=m)
        return cur_v + plsc.all_reduce_population_count(m)

    n_t_v = lax.fori_loop(0, BATCH // 16, scan_body,
                          jnp.zeros((16,), jnp.int32), unroll=4)
    n_t = lax.squeeze(lax.slice(n_t_v, [0], [1]), [0])

    for i in range(_NWIN):
        nxt = fire(i + 1) if i + 1 < _NWIN else None
        cp.wait()
        buf = bufs[i % 2]
        c0 = jnp.minimum(lo + i * _WIN, _C0_MAX)

        def filt_body(g, wcur_v, i=i):
            rv = r_l[pl.ds(g * 16, 16)]
            bv = b_l[pl.ds(g * 16, 16)]
            valid = (g * 16 + iota16) < n_t
            wm = valid & (((rv - lo) >> 11) == i)
            wmi = jnp.where(wm, jnp.int32(1), jnp.int32(0))
            pos = jnp.minimum(wcur_v + jnp.cumsum(wmi) - 1, _CAP - 1)
            plsc.store_scatter(wr_l, [pos], rv, mask=wm)
            plsc.store_scatter(wb_l, [pos], bv, mask=wm)
            return wcur_v + plsc.all_reduce_population_count(wm)

        n_w_v = lax.fori_loop(0, (n_t + 15) >> 4, filt_body,
                              jnp.zeros((16,), jnp.int32))
        n_w = lax.squeeze(lax.slice(n_w_v, [0], [1]), [0])

        def gather_body(g2, _, buf=buf, c0=c0):
            rv = wr_l[pl.ds(g2 * 16, 16)]
            bv = wb_l[pl.ds(g2 * 16, 16)]
            tail = (g2 * 16 + iota16) < n_w
            col = jnp.where(tail, rv - c0, 0)
            for d in range(EMBED_DIM):
                st1[d, :] = plsc.load_gather(
                    buf,
                    [jnp.full((16,), d >> 3, jnp.int32),
                     jnp.full((16,), d & 7, jnp.int32), col])
            for j in range(16):
                st2[j, 0:16] = plsc.load_gather(
                    st1, [iota16, jnp.full((16,), j, jnp.int32)])
            bsafe = jnp.where(tail, bv, _DUMP_ROW)
            pltpu.async_copy(st2, out_hbm.at[bsafe], sem2).wait()
            return 0

        lax.fori_loop(0, (n_w + 15) >> 4, gather_body, 0)
        cp = nxt


@jax.jit
def _embed_lookup(table_t3, idx):
    mesh = plsc.VectorSubcoreMesh(core_axis_name="c", subcore_axis_name="s")
    return pl.kernel(
        _emb_kernel,
        mesh=mesh,
        out_type=jax.ShapeDtypeStruct((_OUT_ROWS, 128), jnp.float32),
        scratch_types=[
            pltpu.VMEM((BATCH,), jnp.int32),
            pltpu.VMEM((_CAP,), jnp.int32),
            pltpu.VMEM((_CAP,), jnp.int32),
            pltpu.VMEM((_CAP,), jnp.int32),
            pltpu.VMEM((_CAP,), jnp.int32),
            pltpu.VMEM((2, 8, _WIN), jnp.float32),
            pltpu.VMEM((2, 8, _WIN), jnp.float32),
            pltpu.VMEM((16, 16), jnp.float32),
            pltpu.VMEM((16, 128), jnp.float32),
            pltpu.SemaphoreType.DMA,
            pltpu.SemaphoreType.DMA,
            pltpu.SemaphoreType.DMA,
        ],
        compiler_params=pltpu.CompilerParams(
            disable_bounds_checks=True, needs_layout_passes=False),
    )(table_t3, idx)


def kernel(class_label, embedding_weight):
    wt3 = embedding_weight.T.reshape(2, 8, N_ROWS)
    out = _embed_lookup(wt3, class_label)
    return out[:BATCH, :EMBED_DIM][:, None, :]
